# baseline (speedup 1.0000x reference)
import functools

import jax
import jax.numpy as jnp
from jax import lax
from jax.experimental import pallas as pl
from jax.experimental.pallas import tpu as pltpu
from jax.experimental.pallas import tpu_sc as plsc

N_NODES = 10000
E = 320000
D = 128
LANES = 16

NC, NS = 2, 16
NW = NC * NS
EPW = E // NW
CHUNK = 80
NCHUNK = EPW // CHUNK
N_PAD = 10240
RPT = N_PAD // NS
SUBR = 64
NSUB = RPT // SUBR
NCH1 = 64
NCH2 = NCHUNK - NCH1

_MESH = plsc.VectorSubcoreMesh(
    core_axis_name="c", subcore_axis_name="s", num_cores=NC, num_subcores=NS
)


def _ab_body(x_ref, w1a_ref, w1b_ref, a_ref, b_ref):
    x = x_ref[...]
    a_ref[...] = jnp.dot(x, w1a_ref[...], preferred_element_type=jnp.float32)
    b_ref[...] = jnp.dot(x, w1b_ref[...], preferred_element_type=jnp.float32)


def _node_proj(node_emb, w1a, w1b):
    blk = 1000
    grid = N_NODES // blk
    return pl.pallas_call(
        _ab_body,
        grid=(grid,),
        in_specs=[
            pl.BlockSpec((blk, D), lambda i: (i, 0)),
            pl.BlockSpec((D, D), lambda i: (0, 0)),
            pl.BlockSpec((D, D), lambda i: (0, 0)),
        ],
        out_specs=[
            pl.BlockSpec((blk, D), lambda i: (i, 0)),
            pl.BlockSpec((blk, D), lambda i: (i, 0)),
        ],
        out_shape=[
            jax.ShapeDtypeStruct((N_NODES, D), jnp.float32),
            jax.ShapeDtypeStruct((N_NODES, D), jnp.float32),
        ],
    )(node_emb, w1a, w1b)


def _make_gather(ch_lo, n_ch):
    epw_h = n_ch * CHUNK
    e_h = NW * epw_h

    @functools.partial(
        pl.kernel,
        out_type=[
            jax.ShapeDtypeStruct((e_h, D), jnp.float32),
            jax.ShapeDtypeStruct((NW, N_PAD), jnp.float32),
        ],
        mesh=_MESH,
        compiler_params=pltpu.CompilerParams(needs_layout_passes=False),
        scratch_types=[
            pltpu.VMEM((epw_h,), jnp.int32),
            pltpu.VMEM((epw_h,), jnp.int32),
            pltpu.VMEM((CHUNK, D), jnp.float32),
            pltpu.VMEM((CHUNK, D), jnp.float32),
            pltpu.VMEM((CHUNK, D), jnp.float32),
            pltpu.VMEM((CHUNK, D), jnp.float32),
            pltpu.VMEM((CHUNK, D), jnp.float32),
            pltpu.VMEM((CHUNK, D), jnp.float32),
            pltpu.VMEM((N_PAD,), jnp.float32),
            pltpu.SemaphoreType.DMA,
            pltpu.SemaphoreType.DMA,
            pltpu.SemaphoreType.DMA,
            pltpu.SemaphoreType.DMA,
            pltpu.SemaphoreType.DMA,
            pltpu.SemaphoreType.DMA,
            pltpu.SemaphoreType.DMA,
            pltpu.SemaphoreType.DMA,
            pltpu.SemaphoreType.DMA,
        ],
    )
    def gather(a_hbm, b_hbm, src_hbm, dst_hbm, dep_hbm, out_hbm, hist_hbm,
               si, di, ba0, bb0, ba1, bb1, ba2, bb2, hist,
               sa0, sb0, sa1, sb1, sa2, sb2, sw0, sw1, sw2):
        del dep_hbm
        wid = lax.axis_index("s") * NC + lax.axis_index("c")
        base_in = wid * EPW + ch_lo * CHUNK
        base_out = wid * epw_h

        bas = (ba0, ba1, ba2)
        bbs = (bb0, bb1, bb2)
        sas = (sa0, sa1, sa2)
        sbs = (sb0, sb1, sb2)
        sws = (sw0, sw1, sw2)

        pltpu.sync_copy(src_hbm.at[pl.ds(base_in, epw_h)], si)
        pltpu.sync_copy(dst_hbm.at[pl.ds(base_in, epw_h)], di)

        zeros16 = jnp.zeros((LANES,), jnp.float32)
        ones16 = jnp.ones((LANES,), jnp.float32)

        @pl.loop(0, N_PAD // LANES)
        def _zh(k):
            hist[pl.ds(k * LANES, LANES)] = zeros16

        def issueg(g, k):
            lo = g * CHUNK
            pltpu.async_copy(a_hbm.at[si.at[pl.ds(lo, CHUNK)]], bas[k], sas[k])
            pltpu.async_copy(b_hbm.at[di.at[pl.ds(lo, CHUNK)]], bbs[k], sbs[k])

        def draing(k):
            pltpu.make_async_copy(a_hbm.at[pl.ds(0, CHUNK)], bas[k], sas[k]).wait()
            pltpu.make_async_copy(b_hbm.at[pl.ds(0, CHUNK)], bbs[k], sbs[k]).wait()

        def proc(g, k):
            lo = g * CHUNK
            ba, bb = bas[k], bbs[k]

            @pl.loop(0, CHUNK, unroll=4)
            def _row(r):
                for j in range(D // LANES):
                    sl = pl.ds(j * LANES, LANES)
                    ba[r, sl] = ba[r, sl] + bb[r, sl]

            for p in range(CHUNK // LANES):
                iv = si[pl.ds(lo + p * LANES, LANES)]
                plsc.addupdate_scatter(hist, [iv], ones16)

        def issuew(g, k):
            pltpu.async_copy(
                bas[k], out_hbm.at[pl.ds(base_out + g * CHUNK, CHUNK)], sws[k]
            )

        def drainw(k):
            pltpu.make_async_copy(
                bas[k], out_hbm.at[pl.ds(base_out, CHUNK)], sws[k]
            ).wait()

        nsteady = (n_ch - 4) // 3
        issueg(0, 0)
        issueg(1, 1)
        draing(0)
        proc(0, 0)
        issuew(0, 0)
        issueg(2, 2)
        draing(1)
        proc(1, 1)
        issuew(1, 1)
        drainw(0)
        issueg(3, 0)

        @pl.loop(0, nsteady)
        def _go(go):
            gb = 2 + go * 3
            for j in range(3):
                g = gb + j
                k = (2 + j) % 3
                kn = (k + 2) % 3
                draing(k)
                proc(g, k)
                issuew(g, k)
                drainw(kn)
                issueg(g + 2, kn)

        for g in (n_ch - 2, n_ch - 1):
            k = g % 3
            draing(k)
            proc(g, k)
            issuew(g, k)
        for g in (n_ch - 3, n_ch - 2, n_ch - 1):
            drainw(g % 3)

        pltpu.sync_copy(hist, hist_hbm.at[wid])

    return gather


_gather1 = _make_gather(0, NCH1)
_gather2 = _make_gather(NCH1, NCH2)


def _mlp_body(h0_ref, b1_ref, w2_ref, b2_ref, wg_ref, bg_ref, out_ref):
    h = jnp.maximum(h0_ref[...] + b1_ref[...], 0.0)
    inter = jnp.dot(h, w2_ref[...], preferred_element_type=jnp.float32) + b2_ref[...]
    gate = jax.nn.sigmoid(
        jnp.dot(inter, wg_ref[...], preferred_element_type=jnp.float32) + bg_ref[...]
    )
    out_ref[...] = inter * gate


def _edge_mlp(h0, b1, w2, b2, wg, bg):
    e_h = h0.shape[0]
    blk = 2560
    grid = e_h // blk
    vec = lambda i: (0, 0)
    return pl.pallas_call(
        _mlp_body,
        grid=(grid,),
        in_specs=[
            pl.BlockSpec((blk, D), lambda i: (i, 0)),
            pl.BlockSpec((1, D), vec),
            pl.BlockSpec((D, D), vec),
            pl.BlockSpec((1, D), vec),
            pl.BlockSpec((D, D), vec),
            pl.BlockSpec((1, D), vec),
        ],
        out_specs=pl.BlockSpec((blk, D), lambda i: (i, 0)),
        out_shape=jax.ShapeDtypeStruct((e_h, D), jnp.float32),
    )(h0, b1.reshape(1, D), w2, b2.reshape(1, D), wg, bg.reshape(1, D))


def _make_scatter(ch_lo, n_ch):
    epw_h = n_ch * CHUNK

    @functools.partial(
        pl.kernel,
        out_type=jax.ShapeDtypeStruct((NC, N_PAD, D), jnp.float32),
        mesh=_MESH,
        scratch_types=[
            pltpu.VMEM((n_ch, CHUNK), jnp.int32),
            pltpu.VMEM((CHUNK, D), jnp.float32),
            pltpu.VMEM((CHUNK, D), jnp.float32),
            pltpu.VMEM((SUBR, D), jnp.float32),
            pltpu.VMEM_SHARED((N_PAD, D), jnp.float32),
            pltpu.SemaphoreType.DMA,
            pltpu.SemaphoreType.DMA,
        ],
    )
    def scatter(gated_hbm, src2d_hbm, dep_hbm, psum_hbm,
                idx_v, rows0, rows1, stage_v, accum, s0, s1):
        del dep_hbm
        cid = lax.axis_index("c")
        sid = lax.axis_index("s")
        wid = sid * NC + cid
        rbase = sid * RPT
        base = wid * epw_h

        pltpu.sync_copy(src2d_hbm.at[wid, pl.ds(ch_lo, n_ch)], idx_v)

        zeros16 = jnp.zeros((LANES,), jnp.float32)

        @pl.loop(0, SUBR)
        def _zero(r):
            for j in range(D // LANES):
                stage_v[r, pl.ds(j * LANES, LANES)] = zeros16

        @pl.loop(0, NSUB)
        def _zinit(k):
            pltpu.sync_copy(stage_v, accum.at[pl.ds(rbase + k * SUBR, SUBR)])

        plsc.subcore_barrier()

        def issue(g, rows, sem):
            pltpu.async_copy(
                gated_hbm.at[pl.ds(base + g * CHUNK, CHUNK)], rows, sem
            )

        def drain(rows, sem):
            pltpu.make_async_copy(
                gated_hbm.at[pl.ds(0, CHUNK)], rows, sem
            ).wait()

        def scat(g, rows):
            pltpu.sync_copy(rows, accum.at[idx_v.at[g]], add=True)

        issue(0, rows0, s0)

        @pl.loop(0, (n_ch - 1) // 2)
        def _go(go):
            g0 = go * 2
            issue(g0 + 1, rows1, s1)
            drain(rows0, s0)
            scat(g0, rows0)
            issue(g0 + 2, rows0, s0)
            drain(rows1, s1)
            scat(g0 + 1, rows1)

        if n_ch % 2 == 0:
            issue(n_ch - 1, rows1, s1)
            drain(rows0, s0)
            scat(n_ch - 2, rows0)
            drain(rows1, s1)
            scat(n_ch - 1, rows1)
        else:
            drain(rows0, s0)
            scat(n_ch - 1, rows0)

        plsc.subcore_barrier()

        @pl.loop(0, NSUB)
        def _wb(k):
            r0 = rbase + k * SUBR
            pltpu.sync_copy(accum.at[pl.ds(r0, SUBR)], stage_v)
            pltpu.sync_copy(stage_v, psum_hbm.at[cid, pl.ds(r0, SUBR)])

    return scatter


_scatter1 = _make_scatter(0, NCH1)
_scatter2 = _make_scatter(NCH1, NCH2)


def _fin_body(p10_ref, p11_ref, p20_ref, p21_ref, h_ref, out_ref):
    ones_w = jnp.ones((2 * NW, 1), jnp.float32)
    cnt = jax.lax.dot_general(
        h_ref[...], ones_w, (((0,), (0,)), ((), ())),
        preferred_element_type=jnp.float32,
    )
    s = (p10_ref[...] + p11_ref[...]) + (p20_ref[...] + p21_ref[...])
    out_ref[...] = s / jnp.maximum(cnt, 1.0)


def _finalize(psum1, psum2, hists):
    blk = 1024
    grid = N_PAD // blk
    blk_spec = pl.BlockSpec((blk, D), lambda i: (i, 0))
    return pl.pallas_call(
        _fin_body,
        grid=(grid,),
        in_specs=[
            blk_spec,
            blk_spec,
            blk_spec,
            blk_spec,
            pl.BlockSpec((2 * NW, blk), lambda i: (0, i)),
        ],
        out_specs=blk_spec,
        out_shape=jax.ShapeDtypeStruct((N_PAD, D), jnp.float32),
    )(psum1[0], psum1[1], psum2[0], psum2[1], hists)


def kernel(node_emb, edge_index, W1, b1, W2, b2, Wg, bg):
    src = edge_index[0]
    dst = edge_index[1]
    src2d = src.reshape(NW, NCHUNK, CHUNK)
    a, b = _node_proj(node_emb, W1[:D], W1[D:])
    h0_1, hist1 = _gather1(a, b, src, dst, a)
    h0_2, hist2 = _gather2(a, b, src, dst, hist1)
    gated1 = _edge_mlp(h0_1, b1, W2, b2, Wg, bg)
    gated2 = _edge_mlp(h0_2, b1, W2, b2, Wg, bg)
    psum1 = _scatter1(gated1, src2d, hist2)
    psum2 = _scatter2(gated2, src2d, psum1)
    hists = jnp.concatenate([hist1, hist2], axis=0)
    return _finalize(psum1, psum2, hists)[:N_NODES]


# speedup vs baseline: 1.3238x; 1.3238x over previous
import functools

import jax
import jax.numpy as jnp
from jax import lax
from jax.experimental import pallas as pl
from jax.experimental.pallas import tpu as pltpu
from jax.experimental.pallas import tpu_sc as plsc

N_NODES = 10000
E = 320000
D = 128
LANES = 16

NC, NS = 2, 16
NW = NC * NS
EPW = E // NW
CHUNK = 80
NCHUNK = EPW // CHUNK
N_PAD = 10240
RPT = N_PAD // NS
SUBR = 64
NSUB = RPT // SUBR
NCH1 = 64
NCH2 = NCHUNK - NCH1

_MESH = plsc.VectorSubcoreMesh(
    core_axis_name="c", subcore_axis_name="s", num_cores=NC, num_subcores=NS
)


def _ab_body(x_ref, w1a_ref, w1b_ref, a_ref, b_ref):
    x = x_ref[...]
    a_ref[...] = jnp.dot(x, w1a_ref[...], preferred_element_type=jnp.float32)
    b_ref[...] = jnp.dot(x, w1b_ref[...], preferred_element_type=jnp.float32)


def _node_proj(node_emb, w1a, w1b):
    blk = 1000
    grid = N_NODES // blk
    return pl.pallas_call(
        _ab_body,
        grid=(grid,),
        in_specs=[
            pl.BlockSpec((blk, D), lambda i: (i, 0)),
            pl.BlockSpec((D, D), lambda i: (0, 0)),
            pl.BlockSpec((D, D), lambda i: (0, 0)),
        ],
        out_specs=[
            pl.BlockSpec((blk, D), lambda i: (i, 0)),
            pl.BlockSpec((blk, D), lambda i: (i, 0)),
        ],
        out_shape=[
            jax.ShapeDtypeStruct((N_NODES, D), jnp.float32),
            jax.ShapeDtypeStruct((N_NODES, D), jnp.float32),
        ],
    )(node_emb, w1a, w1b)


def _make_gather(ch_lo, n_ch):
    epw_h = n_ch * CHUNK
    e_h = NW * epw_h

    @functools.partial(
        pl.kernel,
        out_type=[
            jax.ShapeDtypeStruct((e_h, D), jnp.float32),
            jax.ShapeDtypeStruct((NW, N_PAD), jnp.float32),
        ],
        mesh=_MESH,
        compiler_params=pltpu.CompilerParams(needs_layout_passes=False),
        scratch_types=[
            pltpu.VMEM((epw_h,), jnp.int32),
            pltpu.VMEM((epw_h,), jnp.int32),
            pltpu.VMEM((CHUNK, D), jnp.float32),
            pltpu.VMEM((CHUNK, D), jnp.float32),
            pltpu.VMEM((CHUNK, D), jnp.float32),
            pltpu.VMEM((CHUNK, D), jnp.float32),
            pltpu.VMEM((CHUNK, D), jnp.float32),
            pltpu.VMEM((CHUNK, D), jnp.float32),
            pltpu.VMEM((N_PAD,), jnp.float32),
            pltpu.SemaphoreType.DMA,
            pltpu.SemaphoreType.DMA,
            pltpu.SemaphoreType.DMA,
            pltpu.SemaphoreType.DMA,
            pltpu.SemaphoreType.DMA,
            pltpu.SemaphoreType.DMA,
            pltpu.SemaphoreType.DMA,
            pltpu.SemaphoreType.DMA,
            pltpu.SemaphoreType.DMA,
        ],
    )
    def gather(a_hbm, b_hbm, src_hbm, dst_hbm, dep_hbm, out_hbm, hist_hbm,
               si, di, ba0, bb0, ba1, bb1, ba2, bb2, hist,
               sa0, sb0, sa1, sb1, sa2, sb2, sw0, sw1, sw2):
        del dep_hbm
        wid = lax.axis_index("s") * NC + lax.axis_index("c")
        base_in = wid * EPW + ch_lo * CHUNK
        base_out = wid * epw_h

        bas = (ba0, ba1, ba2)
        bbs = (bb0, bb1, bb2)
        sas = (sa0, sa1, sa2)
        sbs = (sb0, sb1, sb2)
        sws = (sw0, sw1, sw2)

        pltpu.sync_copy(src_hbm.at[pl.ds(base_in, epw_h)], si)
        pltpu.sync_copy(dst_hbm.at[pl.ds(base_in, epw_h)], di)

        zeros16 = jnp.zeros((LANES,), jnp.float32)
        ones16 = jnp.ones((LANES,), jnp.float32)

        @pl.loop(0, N_PAD // LANES)
        def _zh(k):
            hist[pl.ds(k * LANES, LANES)] = zeros16

        def issueg(g, k):
            lo = g * CHUNK
            pltpu.async_copy(a_hbm.at[si.at[pl.ds(lo, CHUNK)]], bas[k], sas[k])
            pltpu.async_copy(b_hbm.at[di.at[pl.ds(lo, CHUNK)]], bbs[k], sbs[k])

        def draing(k):
            pltpu.make_async_copy(a_hbm.at[pl.ds(0, CHUNK)], bas[k], sas[k]).wait()
            pltpu.make_async_copy(b_hbm.at[pl.ds(0, CHUNK)], bbs[k], sbs[k]).wait()

        def proc(g, k):
            lo = g * CHUNK
            ba, bb = bas[k], bbs[k]

            @pl.loop(0, CHUNK)
            def _row(r):
                for j in range(D // LANES):
                    sl = pl.ds(j * LANES, LANES)
                    ba[r, sl] = ba[r, sl] + bb[r, sl]

            for p in range(CHUNK // LANES):
                iv = si[pl.ds(lo + p * LANES, LANES)]
                plsc.addupdate_scatter(hist, [iv], ones16)

        def issuew(g, k):
            pltpu.async_copy(
                bas[k], out_hbm.at[pl.ds(base_out + g * CHUNK, CHUNK)], sws[k]
            )

        def drainw(k):
            pltpu.make_async_copy(
                bas[k], out_hbm.at[pl.ds(base_out, CHUNK)], sws[k]
            ).wait()

        nsteady = (n_ch - 4) // 3
        issueg(0, 0)
        issueg(1, 1)
        draing(0)
        proc(0, 0)
        issuew(0, 0)
        issueg(2, 2)
        draing(1)
        proc(1, 1)
        issuew(1, 1)
        drainw(0)
        issueg(3, 0)

        @pl.loop(0, nsteady)
        def _go(go):
            gb = 2 + go * 3
            for j in range(3):
                g = gb + j
                k = (2 + j) % 3
                kn = (k + 2) % 3
                draing(k)
                proc(g, k)
                issuew(g, k)
                drainw(kn)
                issueg(g + 2, kn)

        for g in (n_ch - 2, n_ch - 1):
            k = g % 3
            draing(k)
            proc(g, k)
            issuew(g, k)
        for g in (n_ch - 3, n_ch - 2, n_ch - 1):
            drainw(g % 3)

        pltpu.sync_copy(hist, hist_hbm.at[wid])

    return gather


_gather1 = _make_gather(0, NCH1)
_gather2 = _make_gather(NCH1, NCH2)


def _mlp_body(h0_ref, b1_ref, w2_ref, b2_ref, wg_ref, bg_ref, out_ref):
    h = jnp.maximum(h0_ref[...] + b1_ref[...], 0.0)
    inter = jnp.dot(h, w2_ref[...], preferred_element_type=jnp.float32) + b2_ref[...]
    gate = jax.nn.sigmoid(
        jnp.dot(inter, wg_ref[...], preferred_element_type=jnp.float32) + bg_ref[...]
    )
    out_ref[...] = inter * gate


def _edge_mlp(h0, b1, w2, b2, wg, bg):
    e_h = h0.shape[0]
    blk = 2560
    grid = e_h // blk
    vec = lambda i: (0, 0)
    return pl.pallas_call(
        _mlp_body,
        grid=(grid,),
        in_specs=[
            pl.BlockSpec((blk, D), lambda i: (i, 0)),
            pl.BlockSpec((1, D), vec),
            pl.BlockSpec((D, D), vec),
            pl.BlockSpec((1, D), vec),
            pl.BlockSpec((D, D), vec),
            pl.BlockSpec((1, D), vec),
        ],
        out_specs=pl.BlockSpec((blk, D), lambda i: (i, 0)),
        out_shape=jax.ShapeDtypeStruct((e_h, D), jnp.float32),
    )(h0, b1.reshape(1, D), w2, b2.reshape(1, D), wg, bg.reshape(1, D))


def _make_scatter(ch_lo, n_ch):
    epw_h = n_ch * CHUNK

    @functools.partial(
        pl.kernel,
        out_type=jax.ShapeDtypeStruct((NC, N_PAD, D), jnp.float32),
        mesh=_MESH,
        scratch_types=[
            pltpu.VMEM((n_ch, CHUNK), jnp.int32),
            pltpu.VMEM((CHUNK, D), jnp.float32),
            pltpu.VMEM((CHUNK, D), jnp.float32),
            pltpu.VMEM((SUBR, D), jnp.float32),
            pltpu.VMEM_SHARED((N_PAD, D), jnp.float32),
            pltpu.SemaphoreType.DMA,
            pltpu.SemaphoreType.DMA,
        ],
    )
    def scatter(gated_hbm, src2d_hbm, dep_hbm, psum_hbm,
                idx_v, rows0, rows1, stage_v, accum, s0, s1):
        del dep_hbm
        cid = lax.axis_index("c")
        sid = lax.axis_index("s")
        wid = sid * NC + cid
        rbase = sid * RPT
        base = wid * epw_h

        pltpu.sync_copy(src2d_hbm.at[wid, pl.ds(ch_lo, n_ch)], idx_v)

        zeros16 = jnp.zeros((LANES,), jnp.float32)

        @pl.loop(0, SUBR)
        def _zero(r):
            for j in range(D // LANES):
                stage_v[r, pl.ds(j * LANES, LANES)] = zeros16

        @pl.loop(0, NSUB)
        def _zinit(k):
            pltpu.sync_copy(stage_v, accum.at[pl.ds(rbase + k * SUBR, SUBR)])

        plsc.subcore_barrier()

        def issue(g, rows, sem):
            pltpu.async_copy(
                gated_hbm.at[pl.ds(base + g * CHUNK, CHUNK)], rows, sem
            )

        def drain(rows, sem):
            pltpu.make_async_copy(
                gated_hbm.at[pl.ds(0, CHUNK)], rows, sem
            ).wait()

        def scat(g, rows):
            pltpu.sync_copy(rows, accum.at[idx_v.at[g]], add=True)

        issue(0, rows0, s0)

        @pl.loop(0, (n_ch - 1) // 2)
        def _go(go):
            g0 = go * 2
            issue(g0 + 1, rows1, s1)
            drain(rows0, s0)
            scat(g0, rows0)
            issue(g0 + 2, rows0, s0)
            drain(rows1, s1)
            scat(g0 + 1, rows1)

        if n_ch % 2 == 0:
            issue(n_ch - 1, rows1, s1)
            drain(rows0, s0)
            scat(n_ch - 2, rows0)
            drain(rows1, s1)
            scat(n_ch - 1, rows1)
        else:
            drain(rows0, s0)
            scat(n_ch - 1, rows0)

        plsc.subcore_barrier()

        @pl.loop(0, NSUB)
        def _wb(k):
            r0 = rbase + k * SUBR
            pltpu.sync_copy(accum.at[pl.ds(r0, SUBR)], stage_v)
            pltpu.sync_copy(stage_v, psum_hbm.at[cid, pl.ds(r0, SUBR)])

    return scatter


_scatter1 = _make_scatter(0, NCH1)
_scatter2 = _make_scatter(NCH1, NCH2)


def _fin_body(p10_ref, p11_ref, p20_ref, p21_ref, h_ref, out_ref):
    ones_w = jnp.ones((2 * NW, 1), jnp.float32)
    cnt = jax.lax.dot_general(
        h_ref[...], ones_w, (((0,), (0,)), ((), ())),
        preferred_element_type=jnp.float32,
    )
    s = (p10_ref[...] + p11_ref[...]) + (p20_ref[...] + p21_ref[...])
    out_ref[...] = s / jnp.maximum(cnt, 1.0)


def _finalize(psum1, psum2, hists):
    blk = 1024
    grid = N_PAD // blk
    blk_spec = pl.BlockSpec((blk, D), lambda i: (i, 0))
    return pl.pallas_call(
        _fin_body,
        grid=(grid,),
        in_specs=[
            blk_spec,
            blk_spec,
            blk_spec,
            blk_spec,
            pl.BlockSpec((2 * NW, blk), lambda i: (0, i)),
        ],
        out_specs=blk_spec,
        out_shape=jax.ShapeDtypeStruct((N_PAD, D), jnp.float32),
    )(psum1[0], psum1[1], psum2[0], psum2[1], hists)


def kernel(node_emb, edge_index, W1, b1, W2, b2, Wg, bg):
    src = edge_index[0]
    dst = edge_index[1]
    src2d = src.reshape(NW, NCHUNK, CHUNK)
    a, b = _node_proj(node_emb, W1[:D], W1[D:])
    h0_1, hist1 = _gather1(a, b, src, dst, a)
    h0_2, hist2 = _gather2(a, b, src, dst, hist1)
    gated1 = _edge_mlp(h0_1, b1, W2, b2, Wg, bg)
    gated2 = _edge_mlp(h0_2, b1, W2, b2, Wg, bg)
    psum1 = _scatter1(gated1, src2d, hist2)
    psum2 = _scatter2(gated2, src2d, psum1)
    hists = jnp.concatenate([hist1, hist2], axis=0)
    return _finalize(psum1, psum2, hists)[:N_NODES]
